# baseline (speedup 1.0000x reference)
import dataclasses
import functools

import jax
import jax.numpy as jnp
from jax import lax
from jax.experimental import pallas as pl
from jax.experimental.pallas import tpu as pltpu
from jax.experimental.pallas import tpu_sc as plsc

N = 10000
E = 320000
D = 128

NC = 2
NS = 16
NW = NC * NS
LANES = 16

C = 128
CH_W = 80
ROWS = NW * CH_W
E_PAD = ROWS * C

DEG_R = 640
N_PAD = 10240
RPT = N_PAD // NS
RSLAB = 128
ESLAB = 16

_mesh = plsc.VectorSubcoreMesh(core_axis_name="c", subcore_axis_name="s")

_sc_params = pltpu.CompilerParams()
if "needs_layout_passes" in pltpu.CompilerParams.__dataclass_fields__:
    _sc_params = dataclasses.replace(_sc_params, needs_layout_passes=False)


DEG_ROWS = 80


@functools.partial(
    pl.kernel,
    out_type=jax.ShapeDtypeStruct((NC, DEG_ROWS, C), jnp.float32),
    mesh=_mesh,
    scratch_types=[
        pltpu.VMEM((CH_W, C), jnp.int32),
        pltpu.VMEM((CH_W, C), jnp.float32),
        pltpu.VMEM((DEG_ROWS, C), jnp.float32),
        pltpu.VMEM((1, DEG_ROWS), jnp.int32),
        pltpu.VMEM_SHARED((DEG_ROWS, C), jnp.float32),
    ],
    compiler_params=_sc_params,
)
def _sc_deg(dst_hbm, ew_hbm, iota_hbm, out_hbm, dstb, ewb, degp, iotab, deg_sh):
    c = lax.axis_index("c")
    s = lax.axis_index("s")
    w = c * NS + s
    zf = jnp.zeros((LANES,), jnp.float32)

    @pl.loop(0, DEG_ROWS)
    def _(r):
        for k in range(C // LANES):
            degp[r, pl.ds(k * LANES, LANES)] = zf

    @pl.when(s < DEG_ROWS // 8)
    def _():
        pltpu.sync_copy(degp.at[pl.ds(0, 8)], deg_sh.at[pl.ds(s * 8, 8)])
    plsc.subcore_barrier()

    pltpu.sync_copy(dst_hbm.at[pl.ds(w * CH_W, CH_W)], dstb)
    pltpu.sync_copy(ew_hbm.at[pl.ds(w * CH_W, CH_W)], ewb)
    pltpu.sync_copy(iota_hbm, iotab)

    @pl.loop(0, CH_W)
    def _(ci):
        for j in range(C // LANES):
            dv = dstb[ci, pl.ds(j * LANES, LANES)]
            ev = ewb[ci, pl.ds(j * LANES, LANES)]
            plsc.addupdate_scatter(degp, [dv >> 7, dv & 127], ev)

    pltpu.sync_copy(degp, deg_sh.at[iotab.at[0]], add=True)
    plsc.subcore_barrier()

    @pl.when(s < DEG_ROWS // 8)
    def _():
        pltpu.sync_copy(deg_sh.at[pl.ds(s * 8, 8)],
                        out_hbm.at[c, pl.ds(s * 8, 8)])


ESLAB = 16
NSLAB = CH_W // ESLAB
HC = 64


NBUF = 4
SLOTS = 2 * ESLAB
NGRP = SLOTS // NBUF


@functools.partial(
    pl.kernel,
    out_type=jax.ShapeDtypeStruct((NC, N_PAD, D), jnp.float32),
    mesh=_mesh,
    scratch_types=[
        pltpu.VMEM((ESLAB, 2, HC), jnp.int32),
        pltpu.VMEM((ESLAB, 2, HC), jnp.int32),
        pltpu.VMEM((ESLAB, 2, HC), jnp.float32),
        pltpu.VMEM((HC, D), jnp.float32),
        pltpu.VMEM((HC, D), jnp.float32),
        pltpu.VMEM((HC, D), jnp.float32),
        pltpu.VMEM((HC, D), jnp.float32),
        pltpu.VMEM_SHARED((N_PAD, D), jnp.float32),
        pltpu.SemaphoreType.DMA,
        pltpu.SemaphoreType.DMA,
        pltpu.SemaphoreType.DMA,
        pltpu.SemaphoreType.DMA,
        pltpu.SemaphoreType.DMA,
        pltpu.SemaphoreType.DMA,
        pltpu.SemaphoreType.DMA,
        pltpu.SemaphoreType.DMA,
    ],
    compiler_params=_sc_params,
)
def _sc_agg(hp_hbm, src_hbm, dst_hbm, ew_hbm, out_hbm,
            srcb, dstb, ewb, buf0, buf1, buf2, buf3, acc,
            gsem0, gsem1, gsem2, gsem3, ssem0, ssem1, ssem2, ssem3):
    c = lax.axis_index("c")
    s = lax.axis_index("s")
    w = c * NS + s
    zf = jnp.zeros((LANES,), jnp.float32)
    bufs = [buf0, buf1, buf2, buf3]
    gsems = [gsem0, gsem1, gsem2, gsem3]
    ssems = [ssem0, ssem1, ssem2, ssem3]

    @pl.loop(0, HC)
    def _(r):
        for k in range(D // LANES):
            buf0[r, pl.ds(k * LANES, LANES)] = zf

    for i in range(RPT // HC):
        pltpu.sync_copy(buf0.at[pl.ds(0, HC)],
                        acc.at[pl.ds(s * RPT + i * HC, HC)])
    plsc.subcore_barrier()

    def issue_gather(ci, h, b):
        pltpu.async_copy(hp_hbm.at[srcb.at[ci, h]], bufs[b], gsems[b])

    def wait_gather(b):
        pltpu.make_async_copy(hp_hbm.at[srcb.at[0, 0]], bufs[b],
                              gsems[b]).wait()

    def issue_scat(ci, h, b):
        pltpu.async_copy(bufs[b], acc.at[dstb.at[ci, h]], ssems[b],
                         add=True)

    def wait_scat(b):
        pltpu.make_async_copy(bufs[b], acc.at[dstb.at[0, 0]],
                              ssems[b]).wait()

    def scale(ci, h, b):
        gbuf = bufs[b]

        @plsc.parallel_loop(0, HC // LANES)
        def _(j):
            evec = ewb[ci, h, pl.ds(j * LANES, LANES)]
            for l in range(LANES):
                e = j * LANES + l
                ew_s = evec[l]
                for k in range(D // LANES):
                    sl = pl.ds(k * LANES, LANES)
                    gbuf[e, sl] = gbuf[e, sl] * ew_s

    @pl.loop(0, NSLAB)
    def _(g):
        row0 = w * CH_W + g * ESLAB
        pltpu.sync_copy(src_hbm.at[pl.ds(row0, ESLAB)], srcb)
        pltpu.sync_copy(dst_hbm.at[pl.ds(row0, ESLAB)], dstb)
        pltpu.sync_copy(ew_hbm.at[pl.ds(row0, ESLAB)], ewb)

        issue_gather(0, 0, 0)
        issue_gather(0, 1, 1)
        issue_gather(1, 0, 2)

        @pl.loop(0, NGRP)
        def _(grp):
            for u in range(NBUF):
                ci = 2 * grp + u // 2
                h = u % 2
                wait_gather(u)
                scale(ci, h, u)
                issue_scat(ci, h, u)
                pv = (u - 1) % NBUF
                t3 = NBUF * grp + u + 3
                if u == 0:
                    @pl.when(grp >= 1)
                    def _():
                        wait_scat(pv)
                    issue_gather(2 * grp + 1, 1, pv)
                else:
                    wait_scat(pv)

                    @pl.when(t3 < SLOTS)
                    def _():
                        issue_gather((t3) // 2, t3 % 2, pv)

        wait_scat(NBUF - 1)

    plsc.subcore_barrier()
    for i in range(RPT // RSLAB):
        r0 = s * RPT + i * RSLAB
        pltpu.sync_copy(acc.at[pl.ds(r0, RSLAB)],
                        out_hbm.at[c, pl.ds(r0, RSLAB)])


_P = jax.lax.Precision.HIGHEST


def _tc1_body(x_ref, w1_ref, degp_ref, hp1_ref, dis_ref):
    deg = degp_ref[0] + degp_ref[1] + 1.0
    dis = jnp.where(deg > 0.0, lax.rsqrt(jnp.maximum(deg, 1e-12)), 0.0)
    h1 = jnp.dot(x_ref[...], w1_ref[...], precision=_P,
                 preferred_element_type=jnp.float32)
    hp1_ref[...] = h1 * dis
    dis_ref[...] = dis


def _tc2_body(a_ref, hp1_ref, dis_ref, b1_ref, w2_ref, hp2_ref):
    dis = dis_ref[...]
    z = dis * (a_ref[0] + a_ref[1] + hp1_ref[...]) + b1_ref[...]
    z = jnp.maximum(z, 0.0)
    h2 = jnp.dot(z, w2_ref[...], precision=_P,
                 preferred_element_type=jnp.float32)
    hp2_ref[...] = h2 * dis


def _tc3_body(a_ref, hp2_ref, dis_ref, b2_ref, out_ref):
    z = dis_ref[...] * (a_ref[0] + a_ref[1] + hp2_ref[...]) + b2_ref[...]
    m = jnp.max(z, axis=1, keepdims=True)
    zs = z - m
    lse = jnp.log(jnp.sum(jnp.exp(zs), axis=1, keepdims=True))
    out_ref[...] = zs - lse


_tc1 = pl.pallas_call(
    _tc1_body,
    out_shape=(jax.ShapeDtypeStruct((N, D), jnp.float32),
               jax.ShapeDtypeStruct((N, 1), jnp.float32)),
)
_tc2 = pl.pallas_call(
    _tc2_body,
    out_shape=jax.ShapeDtypeStruct((N, D), jnp.float32),
)
_tc3 = pl.pallas_call(
    _tc3_body,
    out_shape=jax.ShapeDtypeStruct((N, D), jnp.float32),
)


def kernel(x, edge_index, edge_weight, W1, b1, W2, b2):
    src = edge_index[0]
    dst = edge_index[1]
    pad = E_PAD - E
    zi = jnp.arange(pad, dtype=jnp.int32) % N
    src2 = jnp.concatenate([src.astype(jnp.int32), zi]).reshape(ROWS, C)
    dst2 = jnp.concatenate([dst.astype(jnp.int32), zi]).reshape(ROWS, C)
    ew2 = jnp.concatenate([edge_weight.astype(jnp.float32),
                           jnp.zeros((pad,), jnp.float32)]).reshape(ROWS, C)
    src3 = src2.reshape(ROWS, 2, HC)
    dst3 = dst2.reshape(ROWS, 2, HC)
    ew3 = ew2.reshape(ROWS, 2, HC)
    iota = jnp.arange(DEG_ROWS, dtype=jnp.int32).reshape(1, DEG_ROWS)

    degp = _sc_deg(dst2, ew2, iota)
    deg_col = degp.reshape(NC, DEG_ROWS * C)[:, :N, None]

    hp1, dis = _tc1(x, W1, deg_col)
    agg1 = _sc_agg(hp1, src3, dst3, ew3)[:, :N]
    hp2 = _tc2(agg1, hp1, dis, b1.reshape(1, D), W2)
    agg2 = _sc_agg(hp2, src3, dst3, ew3)[:, :N]
    out = _tc3(agg2, hp2, dis, b2.reshape(1, D))
    return out


# speedup vs baseline: 1.0726x; 1.0726x over previous
import dataclasses
import functools

import jax
import jax.numpy as jnp
from jax import lax
from jax.experimental import pallas as pl
from jax.experimental.pallas import tpu as pltpu
from jax.experimental.pallas import tpu_sc as plsc

N = 10000
E = 320000
D = 128

NC = 2
NS = 16
NW = NC * NS
LANES = 16

C = 128
CH_W = 80
ROWS = NW * CH_W
E_PAD = ROWS * C

DEG_R = 640
N_PAD = 10240
RPT = N_PAD // NS
RSLAB = 128
ESLAB = 16

_mesh = plsc.VectorSubcoreMesh(core_axis_name="c", subcore_axis_name="s")

_sc_params = pltpu.CompilerParams()
if "needs_layout_passes" in pltpu.CompilerParams.__dataclass_fields__:
    _sc_params = dataclasses.replace(_sc_params, needs_layout_passes=False)


DEG_ROWS = 80


@functools.partial(
    pl.kernel,
    out_type=jax.ShapeDtypeStruct((NC, DEG_ROWS, C), jnp.float32),
    mesh=_mesh,
    scratch_types=[
        pltpu.VMEM((CH_W, C), jnp.int32),
        pltpu.VMEM((CH_W, C), jnp.float32),
        pltpu.VMEM((DEG_ROWS, C), jnp.float32),
        pltpu.VMEM((1, DEG_ROWS), jnp.int32),
        pltpu.VMEM_SHARED((DEG_ROWS, C), jnp.float32),
    ],
    compiler_params=_sc_params,
)
def _sc_deg(dst_hbm, ew_hbm, iota_hbm, out_hbm, dstb, ewb, degp, iotab, deg_sh):
    c = lax.axis_index("c")
    s = lax.axis_index("s")
    w = c * NS + s
    zf = jnp.zeros((LANES,), jnp.float32)

    @pl.loop(0, DEG_ROWS)
    def _(r):
        for k in range(C // LANES):
            degp[r, pl.ds(k * LANES, LANES)] = zf

    @pl.when(s < DEG_ROWS // 8)
    def _():
        pltpu.sync_copy(degp.at[pl.ds(0, 8)], deg_sh.at[pl.ds(s * 8, 8)])
    plsc.subcore_barrier()

    pltpu.sync_copy(dst_hbm.at[pl.ds(w * CH_W, CH_W)], dstb)
    pltpu.sync_copy(ew_hbm.at[pl.ds(w * CH_W, CH_W)], ewb)
    pltpu.sync_copy(iota_hbm, iotab)

    @pl.loop(0, CH_W)
    def _(ci):
        for j in range(C // LANES):
            dv = dstb[ci, pl.ds(j * LANES, LANES)]
            ev = ewb[ci, pl.ds(j * LANES, LANES)]
            plsc.addupdate_scatter(degp, [dv >> 7, dv & 127], ev)

    pltpu.sync_copy(degp, deg_sh.at[iotab.at[0]], add=True)
    plsc.subcore_barrier()

    @pl.when(s < DEG_ROWS // 8)
    def _():
        pltpu.sync_copy(deg_sh.at[pl.ds(s * 8, 8)],
                        out_hbm.at[c, pl.ds(s * 8, 8)])


ESLAB = 16
NSLAB = CH_W // ESLAB
HC = 64


NBUF = 4
SLOTS = 2 * ESLAB
NGRP = SLOTS // NBUF


@functools.partial(
    pl.kernel,
    out_type=jax.ShapeDtypeStruct((NC, N_PAD, D), jnp.float32),
    mesh=_mesh,
    scratch_types=[
        pltpu.VMEM((ESLAB, 2, HC), jnp.int32),
        pltpu.VMEM((ESLAB, 2, HC), jnp.int32),
        pltpu.VMEM((ESLAB, 2, HC), jnp.float32),
        pltpu.VMEM((HC, D), jnp.float32),
        pltpu.VMEM((HC, D), jnp.float32),
        pltpu.VMEM((HC, D), jnp.float32),
        pltpu.VMEM((HC, D), jnp.float32),
        pltpu.VMEM_SHARED((N_PAD, D), jnp.float32),
        pltpu.SemaphoreType.DMA,
        pltpu.SemaphoreType.DMA,
        pltpu.SemaphoreType.DMA,
        pltpu.SemaphoreType.DMA,
        pltpu.SemaphoreType.DMA,
        pltpu.SemaphoreType.DMA,
        pltpu.SemaphoreType.DMA,
        pltpu.SemaphoreType.DMA,
    ],
    compiler_params=_sc_params,
)
def _sc_agg(hp_hbm, src_hbm, dst_hbm, ew_hbm, out_hbm,
            srcb, dstb, ewb, buf0, buf1, buf2, buf3, acc,
            gsem0, gsem1, gsem2, gsem3, ssem0, ssem1, ssem2, ssem3):
    c = lax.axis_index("c")
    s = lax.axis_index("s")
    w = c * NS + s
    zf = jnp.zeros((LANES,), jnp.float32)
    bufs = [buf0, buf1, buf2, buf3]
    gsems = [gsem0, gsem1, gsem2, gsem3]
    ssems = [ssem0, ssem1, ssem2, ssem3]

    @pl.loop(0, HC)
    def _(r):
        for k in range(D // LANES):
            buf0[r, pl.ds(k * LANES, LANES)] = zf

    for i in range(RPT // HC):
        pltpu.sync_copy(buf0.at[pl.ds(0, HC)],
                        acc.at[pl.ds(s * RPT + i * HC, HC)])
    plsc.subcore_barrier()

    def issue_gather(ci, h, b):
        pltpu.async_copy(hp_hbm.at[srcb.at[ci, h]], bufs[b], gsems[b])

    def wait_gather(b):
        pltpu.make_async_copy(hp_hbm.at[srcb.at[0, 0]], bufs[b],
                              gsems[b]).wait()

    def issue_scat(ci, h, b):
        pltpu.async_copy(bufs[b], acc.at[dstb.at[ci, h]], ssems[b],
                         add=True)

    def wait_scat(b):
        pltpu.make_async_copy(bufs[b], acc.at[dstb.at[0, 0]],
                              ssems[b]).wait()

    def scale(ci, h, b):
        gbuf = bufs[b]

        @pl.loop(0, HC // LANES, unroll=2)
        def _(j):
            evec = ewb[ci, h, pl.ds(j * LANES, LANES)]
            for l in range(LANES):
                e = j * LANES + l
                ew_s = evec[l]
                for k in range(D // LANES):
                    sl = pl.ds(k * LANES, LANES)
                    gbuf[e, sl] = gbuf[e, sl] * ew_s

    @pl.loop(0, NSLAB)
    def _(g):
        row0 = w * CH_W + g * ESLAB
        pltpu.sync_copy(src_hbm.at[pl.ds(row0, ESLAB)], srcb)
        pltpu.sync_copy(dst_hbm.at[pl.ds(row0, ESLAB)], dstb)
        pltpu.sync_copy(ew_hbm.at[pl.ds(row0, ESLAB)], ewb)

        issue_gather(0, 0, 0)
        issue_gather(0, 1, 1)
        issue_gather(1, 0, 2)

        @pl.loop(0, NGRP)
        def _(grp):
            for u in range(NBUF):
                ci = 2 * grp + u // 2
                h = u % 2
                wait_gather(u)
                scale(ci, h, u)
                issue_scat(ci, h, u)
                pv = (u - 1) % NBUF
                t3 = NBUF * grp + u + 3
                if u == 0:
                    @pl.when(grp >= 1)
                    def _():
                        wait_scat(pv)
                    issue_gather(2 * grp + 1, 1, pv)
                else:
                    wait_scat(pv)

                    @pl.when(t3 < SLOTS)
                    def _():
                        issue_gather((t3) // 2, t3 % 2, pv)

        wait_scat(NBUF - 1)

    plsc.subcore_barrier()
    for i in range(RPT // RSLAB):
        r0 = s * RPT + i * RSLAB
        pltpu.sync_copy(acc.at[pl.ds(r0, RSLAB)],
                        out_hbm.at[c, pl.ds(r0, RSLAB)])


_P = jax.lax.Precision.HIGHEST


def _tc1_body(x_ref, w1_ref, degp_ref, hp1_ref, dis_ref):
    deg = degp_ref[0] + degp_ref[1] + 1.0
    dis = jnp.where(deg > 0.0, lax.rsqrt(jnp.maximum(deg, 1e-12)), 0.0)
    h1 = jnp.dot(x_ref[...], w1_ref[...], precision=_P,
                 preferred_element_type=jnp.float32)
    hp1_ref[...] = h1 * dis
    dis_ref[...] = dis


def _tc2_body(a_ref, hp1_ref, dis_ref, b1_ref, w2_ref, hp2_ref):
    dis = dis_ref[...]
    z = dis * (a_ref[0] + a_ref[1] + hp1_ref[...]) + b1_ref[...]
    z = jnp.maximum(z, 0.0)
    h2 = jnp.dot(z, w2_ref[...], precision=_P,
                 preferred_element_type=jnp.float32)
    hp2_ref[...] = h2 * dis


def _tc3_body(a_ref, hp2_ref, dis_ref, b2_ref, out_ref):
    z = dis_ref[...] * (a_ref[0] + a_ref[1] + hp2_ref[...]) + b2_ref[...]
    m = jnp.max(z, axis=1, keepdims=True)
    zs = z - m
    lse = jnp.log(jnp.sum(jnp.exp(zs), axis=1, keepdims=True))
    out_ref[...] = zs - lse


_tc1 = pl.pallas_call(
    _tc1_body,
    out_shape=(jax.ShapeDtypeStruct((N, D), jnp.float32),
               jax.ShapeDtypeStruct((N, 1), jnp.float32)),
)
_tc2 = pl.pallas_call(
    _tc2_body,
    out_shape=jax.ShapeDtypeStruct((N, D), jnp.float32),
)
_tc3 = pl.pallas_call(
    _tc3_body,
    out_shape=jax.ShapeDtypeStruct((N, D), jnp.float32),
)


def kernel(x, edge_index, edge_weight, W1, b1, W2, b2):
    src = edge_index[0]
    dst = edge_index[1]
    pad = E_PAD - E
    zi = jnp.arange(pad, dtype=jnp.int32) % N
    src2 = jnp.concatenate([src.astype(jnp.int32), zi]).reshape(ROWS, C)
    dst2 = jnp.concatenate([dst.astype(jnp.int32), zi]).reshape(ROWS, C)
    ew2 = jnp.concatenate([edge_weight.astype(jnp.float32),
                           jnp.zeros((pad,), jnp.float32)]).reshape(ROWS, C)
    src3 = src2.reshape(ROWS, 2, HC)
    dst3 = dst2.reshape(ROWS, 2, HC)
    ew3 = ew2.reshape(ROWS, 2, HC)
    iota = jnp.arange(DEG_ROWS, dtype=jnp.int32).reshape(1, DEG_ROWS)

    degp = _sc_deg(dst2, ew2, iota)
    deg_col = degp.reshape(NC, DEG_ROWS * C)[:, :N, None]

    hp1, dis = _tc1(x, W1, deg_col)
    agg1 = _sc_agg(hp1, src3, dst3, ew3)[:, :N]
    hp2 = _tc2(agg1, hp1, dis, b1.reshape(1, D), W2)
    agg2 = _sc_agg(hp2, src3, dst3, ew3)[:, :N]
    out = _tc3(agg2, hp2, dis, b2.reshape(1, D))
    return out
